# baseline (speedup 1.0000x reference)
import jax
import jax.numpy as jnp
from jax.experimental import pallas as pl
from jax.experimental.pallas import tpu as pltpu

_NOPS = 8
_NBATCH = 8


def _gate_body(*refs):
    x_refs = refs[:_NOPS]
    w_ref, b_ref, o_ref, pooled_ref = refs[_NOPS:]
    i = pl.program_id(0)
    for bb in range(_NBATCH):
        parts = [jnp.sum(xq[bb], axis=0, keepdims=True) for xq in x_refs]
        row = parts[0]
        for p in parts[1:]:
            row = row + p
        pooled_ref[pl.ds(i * _NBATCH + bb, 1), :] = row

    @pl.when(i == pl.num_programs(0) - 1)
    def _():
        pooled = pooled_ref[...]
        logits = jax.lax.dot_general(
            pooled, w_ref[...], (((1,), (1,)), ((), ())),
            preferred_element_type=jnp.float32,
        )
        scale = 1.0 / (x_refs[0].shape[1] * _NOPS)
        o_ref[...] = jax.nn.sigmoid(logits * scale + b_ref[...])


def kernel(x, Wc, b):
    B, C, H, W = x.shape
    hw = H * W
    hsl = hw // _NOPS
    xt = jnp.transpose(x, (0, 2, 3, 1)).reshape(B, hw, C)
    b2 = b.reshape(1, C)

    def _xspec(q):
        return pl.BlockSpec((_NBATCH, hsl, C), lambda i, q=q: (i, q, 0))

    out = pl.pallas_call(
        _gate_body,
        grid=(B // _NBATCH,),
        in_specs=[_xspec(q) for q in range(_NOPS)] + [
            pl.BlockSpec((C, C), lambda i: (0, 0)),
            pl.BlockSpec((1, C), lambda i: (0, 0)),
        ],
        out_specs=pl.BlockSpec((B, C), lambda i: (0, 0)),
        out_shape=jax.ShapeDtypeStruct((B, C), jnp.float32),
        scratch_shapes=[pltpu.VMEM((B, C), jnp.float32)],
    )(*([xt] * _NOPS), Wc, b2)
    return out.reshape(B, 1, C, 1, 1)
